# baseline (speedup 1.0000x reference)
import functools

import jax
import jax.numpy as jnp
from jax import lax
from jax.experimental import pallas as pl
from jax.experimental.pallas import tpu as pltpu
from jax.experimental.pallas import tpu_sc as plsc

B = 4
P = 20000
C_ALL = 81
C_FG = 80
TOPK = 200
PRE = 500
CAP = 512
PAD_CAP = 544
CONF_T = 0.98
NMS_T = 0.45
VAR0 = 0.1
VAR1 = 0.2

NC = 2
NS = 16
NW = NC * NS
WPB = NW // B
CPW = C_FG // WPB
LANES = 16


def _decode_body(loc_ref, prior_ref, out_ref):
    cx = prior_ref[0, :][None, :]
    cy = prior_ref[1, :][None, :]
    w = prior_ref[2, :][None, :]
    h = prior_ref[3, :][None, :]
    lx = loc_ref[:, 0, :]
    ly = loc_ref[:, 1, :]
    lw = loc_ref[:, 2, :]
    lh = loc_ref[:, 3, :]
    x = cx + lx * VAR0 * w
    y = cy + ly * VAR0 * h
    wd = w * jnp.exp(lw * VAR1)
    ht = h * jnp.exp(lh * VAR1)
    out_ref[:, 0, :] = x - wd / 2.0
    out_ref[:, 1, :] = y - ht / 2.0
    out_ref[:, 2, :] = x + wd / 2.0
    out_ref[:, 3, :] = y + ht / 2.0


def _decode(loc_t, prior_t):
    return pl.pallas_call(
        _decode_body,
        out_shape=jax.ShapeDtypeStruct((B, 4, P), jnp.float32),
    )(loc_t, prior_t)


def _sc_body(conf_hbm, dec_hbm, s_out, x1_out, y1_out, x2_out, y2_out, cnt_out,
             dec_v, sc_v, cs_v, ci_v, cc_v, cn_v):
    wid = lax.axis_index("s") * NC + lax.axis_index("c")
    b = wid // WPB
    s8 = wid % WPB

    pltpu.sync_copy(dec_hbm.at[b], dec_v)

    lane_iota = lax.iota(jnp.int32, LANES)
    zf16 = jnp.zeros((LANES,), jnp.float32)
    zi16 = jnp.zeros((LANES,), jnp.int32)
    counts_vec = zi16

    for j in range(CPW):
        c = s8 * CPW + j
        pltpu.sync_copy(conf_hbm.at[b, c], sc_v)

        def _zero(g, _):
            off = pl.multiple_of(g * LANES, LANES)
            cs_v[pl.ds(off, LANES)] = zf16
            ci_v[pl.ds(off, LANES)] = zi16
            return 0

        lax.fori_loop(0, PAD_CAP // LANES, _zero, 0)

        def _compact(k, off):
            src = pl.multiple_of(k * LANES, LANES)
            v = sc_v[pl.ds(src, LANES)]
            m = v > CONF_T

            def _hit(off):
                mi = m.astype(jnp.int32)
                c1 = plsc.cumsum(mi)
                pos = jnp.minimum(off + c1 - 1, PAD_CAP - 1)
                plsc.store_scatter(cs_v, [pos], v, mask=m)
                plsc.store_scatter(ci_v, [pos], lane_iota + src, mask=m)
                return off + jnp.sum(mi)

            return lax.cond(jnp.any(m), _hit, lambda off: off, off)

        cnt_c = lax.fori_loop(0, P // LANES, _compact, jnp.int32(0))

        def _gather(g, _):
            off = pl.multiple_of(g * LANES, LANES)
            idxs = ci_v[pl.ds(off, LANES)]
            for p in range(4):
                pv = plsc.load_gather(
                    dec_v, [jnp.full((LANES,), p, jnp.int32), idxs])
                cc_v[p, pl.ds(off, LANES)] = pv
            return 0

        lax.fori_loop(0, CAP // LANES, _gather, 0)

        pltpu.sync_copy(cs_v.at[pl.ds(0, CAP)], s_out.at[b, c])
        pltpu.sync_copy(cc_v.at[0, pl.ds(0, CAP)], x1_out.at[b, c])
        pltpu.sync_copy(cc_v.at[1, pl.ds(0, CAP)], y1_out.at[b, c])
        pltpu.sync_copy(cc_v.at[2, pl.ds(0, CAP)], x2_out.at[b, c])
        pltpu.sync_copy(cc_v.at[3, pl.ds(0, CAP)], y2_out.at[b, c])

        counts_vec = counts_vec + jnp.where(lane_iota == j, cnt_c, 0)

    cn_v[...] = counts_vec
    pltpu.sync_copy(cn_v, cnt_out.at[b, s8])


def _sc_compact(conf_t, dec):
    buf = jax.ShapeDtypeStruct((B, C_FG, CAP), jnp.float32)
    run = pl.kernel(
        _sc_body,
        out_type=[buf, buf, buf, buf, buf,
                  jax.ShapeDtypeStruct((B, WPB, LANES), jnp.int32)],
        mesh=plsc.VectorSubcoreMesh(core_axis_name="c", subcore_axis_name="s"),
        compiler_params=pltpu.CompilerParams(needs_layout_passes=False),
        scratch_types=[
            pltpu.VMEM((4, P), jnp.float32),
            pltpu.VMEM((P,), jnp.float32),
            pltpu.VMEM((PAD_CAP,), jnp.float32),
            pltpu.VMEM((PAD_CAP,), jnp.int32),
            pltpu.VMEM((4, PAD_CAP), jnp.float32),
            pltpu.VMEM((LANES,), jnp.int32),
        ],
    )
    return run(conf_t, dec)


def _nms_body(s_ref, x1_ref, y1_ref, x2_ref, y2_ref, cnt_ref,
              os_ref, ox1_ref, oy1_ref, ox2_ref, oy2_ref):
    R = B * C_FG
    s0 = s_ref[...]
    x1 = x1_ref[...]
    y1 = y1_ref[...]
    x2 = x2_ref[...]
    y2 = y2_ref[...]
    cnt = cnt_ref[...]

    need = cnt > PRE

    def _bs(_, lohi):
        lo, hi = lohi
        mid = (lo + hi) * 0.5
        cgt = jnp.sum((s0 > mid).astype(jnp.int32), axis=1, keepdims=True)
        ge = cgt >= PRE
        return jnp.where(ge, mid, lo), jnp.where(ge, hi, mid)

    lo, _ = lax.fori_loop(
        0, 30, _bs,
        (jnp.full((R, 1), CONF_T, jnp.float32),
         jnp.full((R, 1), 1.0, jnp.float32)))
    s0 = jnp.where(need & (s0 <= lo), 0.0, s0)

    area = jnp.maximum(x2 - x1, 0.0) * jnp.maximum(y2 - y1, 0.0)
    iota2 = lax.broadcasted_iota(jnp.int32, (R, CAP), 1)
    iota_t = lax.broadcasted_iota(jnp.int32, (R, TOPK), 1)
    zout = jnp.zeros((R, TOPK), jnp.float32)

    def _step(t, carry):
        s, o_s, o_x1, o_y1, o_x2, o_y2 = carry
        best = jnp.max(s, axis=1, keepdims=True)
        iseq = s == best
        idx = jnp.min(jnp.where(iseq, iota2, CAP), axis=1, keepdims=True)
        onehot = iseq & (iota2 == idx)
        ohf = onehot.astype(jnp.float32)
        bx1 = jnp.sum(x1 * ohf, axis=1, keepdims=True)
        by1 = jnp.sum(y1 * ohf, axis=1, keepdims=True)
        bx2 = jnp.sum(x2 * ohf, axis=1, keepdims=True)
        by2 = jnp.sum(y2 * ohf, axis=1, keepdims=True)
        barea = jnp.maximum(bx2 - bx1, 0.0) * jnp.maximum(by2 - by1, 0.0)

        ltx = jnp.maximum(x1, bx1)
        lty = jnp.maximum(y1, by1)
        rbx = jnp.minimum(x2, bx2)
        rby = jnp.minimum(y2, by2)
        inter = jnp.maximum(rbx - ltx, 0.0) * jnp.maximum(rby - lty, 0.0)
        union = barea + area - inter
        iou = inter / jnp.maximum(union, 1e-9)
        sup = iou > NMS_T

        valid = best > 0.0
        tm = iota_t == t
        o_s = jnp.where(tm & valid, best, o_s)
        o_x1 = jnp.where(tm & valid, bx1, o_x1)
        o_y1 = jnp.where(tm & valid, by1, o_y1)
        o_x2 = jnp.where(tm & valid, bx2, o_x2)
        o_y2 = jnp.where(tm & valid, by2, o_y2)
        s = jnp.where(sup | onehot, 0.0, s)
        return s, o_s, o_x1, o_y1, o_x2, o_y2

    _, o_s, o_x1, o_y1, o_x2, o_y2 = lax.fori_loop(
        0, TOPK, _step, (s0, zout, zout, zout, zout, zout))
    os_ref[...] = o_s
    ox1_ref[...] = o_x1
    oy1_ref[...] = o_y1
    ox2_ref[...] = o_x2
    oy2_ref[...] = o_y2


def _nms(s2, x12, y12, x22, y22, cnt2):
    out = jax.ShapeDtypeStruct((B * C_FG, TOPK), jnp.float32)
    return pl.pallas_call(
        _nms_body,
        out_shape=[out, out, out, out, out],
    )(s2, x12, y12, x22, y22, cnt2)


def kernel(loc_data, conf_data, prior_data):
    loc_t = jnp.transpose(loc_data, (0, 2, 1))
    prior_t = jnp.transpose(prior_data, (1, 0))
    conf_t = jnp.transpose(
        conf_data.reshape(B, P, C_ALL), (0, 2, 1))[:, 1:, :]

    dec = _decode(loc_t, prior_t)
    s_c, x1_c, y1_c, x2_c, y2_c, cnts = _sc_compact(conf_t, dec)

    cnt2 = cnts[:, :, :CPW].reshape(B * C_FG, 1)
    outs = _nms(
        s_c.reshape(B * C_FG, CAP),
        x1_c.reshape(B * C_FG, CAP),
        y1_c.reshape(B * C_FG, CAP),
        x2_c.reshape(B * C_FG, CAP),
        y2_c.reshape(B * C_FG, CAP),
        cnt2,
    )
    planes = [o.reshape(B, C_FG, TOPK) for o in outs]
    per = jnp.stack(planes, axis=-1)
    bg = jnp.zeros((B, 1, TOPK, 5), jnp.float32)
    return jnp.concatenate([bg, per], axis=1)


# speedup vs baseline: 1.5024x; 1.5024x over previous
import functools

import jax
import jax.numpy as jnp
from jax import lax
from jax.experimental import pallas as pl
from jax.experimental.pallas import tpu as pltpu
from jax.experimental.pallas import tpu_sc as plsc

B = 4
P = 20000
C_ALL = 81
C_FG = 80
TOPK = 200
PRE = 500
CAP = 512
PAD_CAP = 544
CONF_T = 0.98
NMS_T = 0.45
VAR0 = 0.1
VAR1 = 0.2

NC = 2
NS = 16
NW = NC * NS
WPB = NW // B
CPW = C_FG // WPB
LANES = 16


def _decode_body(loc_ref, prior_ref, conf_ref, out_ref, conft_ref):
    cx = prior_ref[0, :]
    cy = prior_ref[1, :]
    w = prior_ref[2, :]
    h = prior_ref[3, :]
    lx = loc_ref[0, 0, :]
    ly = loc_ref[0, 1, :]
    lw = loc_ref[0, 2, :]
    lh = loc_ref[0, 3, :]
    x = cx + lx * VAR0 * w
    y = cy + ly * VAR0 * h
    wd = w * jnp.exp(lw * VAR1)
    ht = h * jnp.exp(lh * VAR1)
    out_ref[0, 0, :] = x - wd / 2.0
    out_ref[0, 1, :] = y - ht / 2.0
    out_ref[0, 2, :] = x + wd / 2.0
    out_ref[0, 3, :] = y + ht / 2.0
    conft_ref[0] = jnp.transpose(conf_ref[0], (1, 0))[1:, :]


def _decode(loc_t, prior_t, conf3):
    return pl.pallas_call(
        _decode_body,
        grid=(B,),
        in_specs=[
            pl.BlockSpec((1, 4, P), lambda b: (b, 0, 0)),
            pl.BlockSpec((4, P), lambda b: (0, 0)),
            pl.BlockSpec((1, P, C_ALL), lambda b: (b, 0, 0)),
        ],
        out_specs=[
            pl.BlockSpec((1, 4, P), lambda b: (b, 0, 0)),
            pl.BlockSpec((1, C_FG, P), lambda b: (b, 0, 0)),
        ],
        out_shape=[
            jax.ShapeDtypeStruct((B, 4, P), jnp.float32),
            jax.ShapeDtypeStruct((B, C_FG, P), jnp.float32),
        ],
    )(loc_t, prior_t, conf3)


def _sc_body(conf_hbm, dec_hbm, s_out, x1_out, y1_out, x2_out, y2_out, cnt_out,
             dec_v, sc_v, cs_v, ci_v, cc_v, cn_v):
    wid = lax.axis_index("s") * NC + lax.axis_index("c")
    b = wid // WPB
    s8 = wid % WPB

    pltpu.sync_copy(dec_hbm.at[b], dec_v)

    lane_iota = lax.iota(jnp.int32, LANES)
    zf16 = jnp.zeros((LANES,), jnp.float32)
    zi16 = jnp.zeros((LANES,), jnp.int32)
    counts_vec = zi16

    for j in range(CPW):
        c = s8 * CPW + j
        pltpu.sync_copy(conf_hbm.at[b, c], sc_v)

        def _zero(g, _):
            off = pl.multiple_of(g * LANES, LANES)
            cs_v[pl.ds(off, LANES)] = zf16
            ci_v[pl.ds(off, LANES)] = zi16
            return 0

        lax.fori_loop(0, PAD_CAP // LANES, _zero, 0)

        def _compact(k, off):
            src = pl.multiple_of(k * LANES, LANES)
            v = sc_v[pl.ds(src, LANES)]
            m = v > CONF_T
            mi = m.astype(jnp.int32)
            c1 = plsc.cumsum(mi)
            pos = jnp.minimum(off + c1 - 1, PAD_CAP - 1)
            plsc.store_scatter(cs_v, [pos], v, mask=m)
            plsc.store_scatter(ci_v, [pos], lane_iota + src, mask=m)
            return off + jnp.sum(mi)

        cnt_c = lax.fori_loop(0, P // LANES, _compact, jnp.int32(0))

        def _gather(g, _):
            off = pl.multiple_of(g * LANES, LANES)
            idxs = ci_v[pl.ds(off, LANES)]
            for p in range(4):
                pv = plsc.load_gather(
                    dec_v, [jnp.full((LANES,), p, jnp.int32), idxs])
                cc_v[p, pl.ds(off, LANES)] = pv
            return 0

        lax.fori_loop(0, CAP // LANES, _gather, 0)

        pltpu.sync_copy(cs_v.at[pl.ds(0, CAP)], s_out.at[b, c])
        pltpu.sync_copy(cc_v.at[0, pl.ds(0, CAP)], x1_out.at[b, c])
        pltpu.sync_copy(cc_v.at[1, pl.ds(0, CAP)], y1_out.at[b, c])
        pltpu.sync_copy(cc_v.at[2, pl.ds(0, CAP)], x2_out.at[b, c])
        pltpu.sync_copy(cc_v.at[3, pl.ds(0, CAP)], y2_out.at[b, c])

        counts_vec = counts_vec + jnp.where(lane_iota == j, cnt_c, 0)

    cn_v[...] = counts_vec
    pltpu.sync_copy(cn_v, cnt_out.at[b, s8])


def _sc_compact(conf_t, dec):
    buf = jax.ShapeDtypeStruct((B, C_FG, CAP), jnp.float32)
    run = pl.kernel(
        _sc_body,
        out_type=[buf, buf, buf, buf, buf,
                  jax.ShapeDtypeStruct((B, WPB, LANES), jnp.int32)],
        mesh=plsc.VectorSubcoreMesh(core_axis_name="c", subcore_axis_name="s"),
        compiler_params=pltpu.CompilerParams(needs_layout_passes=False),
        scratch_types=[
            pltpu.VMEM((4, P), jnp.float32),
            pltpu.VMEM((P,), jnp.float32),
            pltpu.VMEM((PAD_CAP,), jnp.float32),
            pltpu.VMEM((PAD_CAP,), jnp.int32),
            pltpu.VMEM((4, PAD_CAP), jnp.float32),
            pltpu.VMEM((LANES,), jnp.int32),
        ],
    )
    return run(conf_t, dec)


def _nms_body(s_ref, x1_ref, y1_ref, x2_ref, y2_ref, cnt_ref,
              os_ref, ox1_ref, oy1_ref, ox2_ref, oy2_ref):
    R = B * C_FG
    s0 = s_ref[...]
    x1 = x1_ref[...]
    y1 = y1_ref[...]
    x2 = x2_ref[...]
    y2 = y2_ref[...]
    cnt = cnt_ref[...]

    need = cnt > PRE

    def _bs(_, lohi):
        lo, hi = lohi
        mid = (lo + hi) * 0.5
        cgt = jnp.sum((s0 > mid).astype(jnp.int32), axis=1, keepdims=True)
        ge = cgt >= PRE
        return jnp.where(ge, mid, lo), jnp.where(ge, hi, mid)

    lo, _ = lax.fori_loop(
        0, 30, _bs,
        (jnp.full((R, 1), CONF_T, jnp.float32),
         jnp.full((R, 1), 1.0, jnp.float32)))
    s0 = jnp.where(need & (s0 <= lo), 0.0, s0)

    area = jnp.maximum(x2 - x1, 0.0) * jnp.maximum(y2 - y1, 0.0)
    iota2 = lax.broadcasted_iota(jnp.int32, (R, CAP), 1)
    iota_t = lax.broadcasted_iota(jnp.int32, (R, TOPK), 1)
    zout = jnp.zeros((R, TOPK), jnp.float32)

    def _step(t, carry):
        s, o_s, o_x1, o_y1, o_x2, o_y2 = carry
        best = jnp.max(s, axis=1, keepdims=True)
        iseq = s == best
        idx = jnp.min(jnp.where(iseq, iota2, CAP), axis=1, keepdims=True)
        onehot = iseq & (iota2 == idx)
        ohf = onehot.astype(jnp.float32)
        bx1 = jnp.sum(x1 * ohf, axis=1, keepdims=True)
        by1 = jnp.sum(y1 * ohf, axis=1, keepdims=True)
        bx2 = jnp.sum(x2 * ohf, axis=1, keepdims=True)
        by2 = jnp.sum(y2 * ohf, axis=1, keepdims=True)
        barea = jnp.maximum(bx2 - bx1, 0.0) * jnp.maximum(by2 - by1, 0.0)

        ltx = jnp.maximum(x1, bx1)
        lty = jnp.maximum(y1, by1)
        rbx = jnp.minimum(x2, bx2)
        rby = jnp.minimum(y2, by2)
        inter = jnp.maximum(rbx - ltx, 0.0) * jnp.maximum(rby - lty, 0.0)
        union = barea + area - inter
        iou = inter / jnp.maximum(union, 1e-9)
        sup = iou > NMS_T

        valid = best > 0.0
        tm = iota_t == t
        o_s = jnp.where(tm & valid, best, o_s)
        o_x1 = jnp.where(tm & valid, bx1, o_x1)
        o_y1 = jnp.where(tm & valid, by1, o_y1)
        o_x2 = jnp.where(tm & valid, bx2, o_x2)
        o_y2 = jnp.where(tm & valid, by2, o_y2)
        s = jnp.where(sup | onehot, 0.0, s)
        return s, o_s, o_x1, o_y1, o_x2, o_y2

    _, o_s, o_x1, o_y1, o_x2, o_y2 = lax.fori_loop(
        0, TOPK, _step, (s0, zout, zout, zout, zout, zout))
    os_ref[...] = o_s
    ox1_ref[...] = o_x1
    oy1_ref[...] = o_y1
    ox2_ref[...] = o_x2
    oy2_ref[...] = o_y2


def _nms(s2, x12, y12, x22, y22, cnt2):
    out = jax.ShapeDtypeStruct((B * C_FG, TOPK), jnp.float32)
    return pl.pallas_call(
        _nms_body,
        out_shape=[out, out, out, out, out],
    )(s2, x12, y12, x22, y22, cnt2)


def kernel(loc_data, conf_data, prior_data):
    loc_t = jnp.transpose(loc_data, (0, 2, 1))
    prior_t = jnp.transpose(prior_data, (1, 0))
    conf3 = conf_data.reshape(B, P, C_ALL)

    dec, conf_t = _decode(loc_t, prior_t, conf3)
    s_c, x1_c, y1_c, x2_c, y2_c, cnts = _sc_compact(conf_t, dec)

    cnt2 = cnts[:, :, :CPW].reshape(B * C_FG, 1)
    outs = _nms(
        s_c.reshape(B * C_FG, CAP),
        x1_c.reshape(B * C_FG, CAP),
        y1_c.reshape(B * C_FG, CAP),
        x2_c.reshape(B * C_FG, CAP),
        y2_c.reshape(B * C_FG, CAP),
        cnt2,
    )
    planes = [o.reshape(B, C_FG, TOPK) for o in outs]
    per = jnp.stack(planes, axis=-1)
    bg = jnp.zeros((B, 1, TOPK, 5), jnp.float32)
    return jnp.concatenate([bg, per], axis=1)


# speedup vs baseline: 1.7426x; 1.1599x over previous
import functools

import jax
import jax.numpy as jnp
from jax import lax
from jax.experimental import pallas as pl
from jax.experimental.pallas import tpu as pltpu
from jax.experimental.pallas import tpu_sc as plsc

B = 4
P = 20000
C_ALL = 81
C_FG = 80
TOPK = 200
PRE = 500
CAP = 512
CLAMP = 527
PAD_CAP = 576
CONF_T = 0.98
NMS_T = 0.45
VAR0 = 0.1
VAR1 = 0.2

NC = 2
NS = 16
NW = NC * NS
WPB = NW // B
CPW = C_FG // WPB
LANES = 16


def _decode_body(loc_ref, prior_ref, conf_ref, out_ref, conft_ref):
    cx = prior_ref[0, :]
    cy = prior_ref[1, :]
    w = prior_ref[2, :]
    h = prior_ref[3, :]
    lx = loc_ref[0, 0, :]
    ly = loc_ref[0, 1, :]
    lw = loc_ref[0, 2, :]
    lh = loc_ref[0, 3, :]
    x = cx + lx * VAR0 * w
    y = cy + ly * VAR0 * h
    wd = w * jnp.exp(lw * VAR1)
    ht = h * jnp.exp(lh * VAR1)
    out_ref[0, 0, :] = x - wd / 2.0
    out_ref[0, 1, :] = y - ht / 2.0
    out_ref[0, 2, :] = x + wd / 2.0
    out_ref[0, 3, :] = y + ht / 2.0
    conft_ref[0] = jnp.transpose(conf_ref[0], (1, 0))[1:, :]


def _decode(loc_t, prior_t, conf3):
    return pl.pallas_call(
        _decode_body,
        grid=(B,),
        in_specs=[
            pl.BlockSpec((1, 4, P), lambda b: (b, 0, 0)),
            pl.BlockSpec((4, P), lambda b: (0, 0)),
            pl.BlockSpec((1, P, C_ALL), lambda b: (b, 0, 0)),
        ],
        out_specs=[
            pl.BlockSpec((1, 4, P), lambda b: (b, 0, 0)),
            pl.BlockSpec((1, C_FG, P), lambda b: (b, 0, 0)),
        ],
        out_shape=[
            jax.ShapeDtypeStruct((B, 4, P), jnp.float32),
            jax.ShapeDtypeStruct((B, C_FG, P), jnp.float32),
        ],
    )(loc_t, prior_t, conf3)


def _sc_body(conf_hbm, dec_hbm, s_out, x1_out, y1_out, x2_out, y2_out, cnt_out,
             dec_v, sc_v, cs_v, ci_v, cc_v, cn_v):
    wid = lax.axis_index("s") * NC + lax.axis_index("c")
    b = wid // WPB
    s8 = wid % WPB

    pltpu.sync_copy(dec_hbm.at[b], dec_v)

    lane_iota = lax.iota(jnp.int32, LANES)
    zf16 = jnp.zeros((LANES,), jnp.float32)
    zi16 = jnp.zeros((LANES,), jnp.int32)
    counts_vec = zi16

    for j in range(CPW):
        c = s8 * CPW + j
        pltpu.sync_copy(conf_hbm.at[b, c], sc_v)

        def _zero(g, _):
            off = pl.multiple_of(g * LANES, LANES)
            cs_v[pl.ds(off, LANES)] = zf16
            ci_v[pl.ds(off, LANES)] = zi16
            return 0

        lax.fori_loop(0, PAD_CAP // LANES, _zero, 0)

        def _compact(k, carry):
            offm1, piv = carry
            src = pl.multiple_of(k * (2 * LANES), 2 * LANES)
            v1 = sc_v[pl.ds(src, LANES)]
            v2 = sc_v[pl.ds(src + LANES, LANES)]
            m1 = v1 > CONF_T
            m2 = v2 > CONF_T
            c11 = plsc.cumsum(m1.astype(jnp.int32))
            c12 = plsc.cumsum(m2.astype(jnp.int32))
            pc1 = plsc.all_reduce_population_count(m1)
            pc2 = plsc.all_reduce_population_count(m2)
            offc = jnp.minimum(offm1, CLAMP)
            pos1 = offc + c11
            plsc.store_scatter(cs_v, [pos1], v1, mask=m1)
            plsc.store_scatter(ci_v, [pos1], piv, mask=m1)
            pos2 = offc + pc1 + c12
            plsc.store_scatter(cs_v, [pos2], v2, mask=m2)
            plsc.store_scatter(ci_v, [pos2], piv + LANES, mask=m2)
            return offm1 + (pc1 + pc2), piv + 2 * LANES

        offm1, _ = lax.fori_loop(
            0, P // (2 * LANES), _compact,
            (jnp.full((LANES,), -1, jnp.int32), lane_iota))
        cnt_vec = offm1 + 1

        def _gather(g, _):
            off = pl.multiple_of(g * LANES, LANES)
            idxs = ci_v[pl.ds(off, LANES)]
            for p in range(4):
                pv = plsc.load_gather(
                    dec_v, [jnp.full((LANES,), p, jnp.int32), idxs])
                cc_v[p, pl.ds(off, LANES)] = pv
            return 0

        lax.fori_loop(0, CAP // LANES, _gather, 0)

        pltpu.sync_copy(cs_v.at[pl.ds(0, CAP)], s_out.at[b, c])
        pltpu.sync_copy(cc_v.at[0, pl.ds(0, CAP)], x1_out.at[b, c])
        pltpu.sync_copy(cc_v.at[1, pl.ds(0, CAP)], y1_out.at[b, c])
        pltpu.sync_copy(cc_v.at[2, pl.ds(0, CAP)], x2_out.at[b, c])
        pltpu.sync_copy(cc_v.at[3, pl.ds(0, CAP)], y2_out.at[b, c])

        counts_vec = counts_vec + jnp.where(lane_iota == j, cnt_vec, 0)

    cn_v[...] = counts_vec
    pltpu.sync_copy(cn_v, cnt_out.at[b, s8])


def _sc_compact(conf_t, dec):
    buf = jax.ShapeDtypeStruct((B, C_FG, CAP), jnp.float32)
    run = pl.kernel(
        _sc_body,
        out_type=[buf, buf, buf, buf, buf,
                  jax.ShapeDtypeStruct((B, WPB, LANES), jnp.int32)],
        mesh=plsc.VectorSubcoreMesh(core_axis_name="c", subcore_axis_name="s"),
        compiler_params=pltpu.CompilerParams(needs_layout_passes=False),
        scratch_types=[
            pltpu.VMEM((4, P), jnp.float32),
            pltpu.VMEM((P,), jnp.float32),
            pltpu.VMEM((PAD_CAP,), jnp.float32),
            pltpu.VMEM((PAD_CAP,), jnp.int32),
            pltpu.VMEM((4, PAD_CAP), jnp.float32),
            pltpu.VMEM((LANES,), jnp.int32),
        ],
    )
    return run(conf_t, dec)


def _nms_body(s_ref, x1_ref, y1_ref, x2_ref, y2_ref, cnt_ref,
              os_ref, ox1_ref, oy1_ref, ox2_ref, oy2_ref):
    R = B * C_FG
    s0 = s_ref[...]
    x1 = x1_ref[...]
    y1 = y1_ref[...]
    x2 = x2_ref[...]
    y2 = y2_ref[...]
    cnt = cnt_ref[...]

    need = cnt > PRE

    def _bs(_, lohi):
        lo, hi = lohi
        mid = (lo + hi) * 0.5
        cgt = jnp.sum((s0 > mid).astype(jnp.int32), axis=1, keepdims=True)
        ge = cgt >= PRE
        return jnp.where(ge, mid, lo), jnp.where(ge, hi, mid)

    lo, _ = lax.fori_loop(
        0, 30, _bs,
        (jnp.full((R, 1), CONF_T, jnp.float32),
         jnp.full((R, 1), 1.0, jnp.float32)))
    s0 = jnp.where(need & (s0 <= lo), 0.0, s0)

    area = jnp.maximum(x2 - x1, 0.0) * jnp.maximum(y2 - y1, 0.0)
    iota2 = lax.broadcasted_iota(jnp.int32, (R, CAP), 1)
    iota_t = lax.broadcasted_iota(jnp.int32, (R, TOPK), 1)
    zout = jnp.zeros((R, TOPK), jnp.float32)

    def _step(t, carry):
        s, o_s, o_x1, o_y1, o_x2, o_y2 = carry
        best = jnp.max(s, axis=1, keepdims=True)
        iseq = s == best
        idx = jnp.min(jnp.where(iseq, iota2, CAP), axis=1, keepdims=True)
        onehot = iseq & (iota2 == idx)
        ohf = onehot.astype(jnp.float32)
        bx1 = jnp.sum(x1 * ohf, axis=1, keepdims=True)
        by1 = jnp.sum(y1 * ohf, axis=1, keepdims=True)
        bx2 = jnp.sum(x2 * ohf, axis=1, keepdims=True)
        by2 = jnp.sum(y2 * ohf, axis=1, keepdims=True)
        barea = jnp.maximum(bx2 - bx1, 0.0) * jnp.maximum(by2 - by1, 0.0)

        ltx = jnp.maximum(x1, bx1)
        lty = jnp.maximum(y1, by1)
        rbx = jnp.minimum(x2, bx2)
        rby = jnp.minimum(y2, by2)
        inter = jnp.maximum(rbx - ltx, 0.0) * jnp.maximum(rby - lty, 0.0)
        union = barea + area - inter
        iou = inter / jnp.maximum(union, 1e-9)
        sup = iou > NMS_T

        valid = best > 0.0
        tm = iota_t == t
        o_s = jnp.where(tm & valid, best, o_s)
        o_x1 = jnp.where(tm & valid, bx1, o_x1)
        o_y1 = jnp.where(tm & valid, by1, o_y1)
        o_x2 = jnp.where(tm & valid, bx2, o_x2)
        o_y2 = jnp.where(tm & valid, by2, o_y2)
        s = jnp.where(sup | onehot, 0.0, s)
        return s, o_s, o_x1, o_y1, o_x2, o_y2

    _, o_s, o_x1, o_y1, o_x2, o_y2 = lax.fori_loop(
        0, TOPK, _step, (s0, zout, zout, zout, zout, zout))
    os_ref[...] = o_s
    ox1_ref[...] = o_x1
    oy1_ref[...] = o_y1
    ox2_ref[...] = o_x2
    oy2_ref[...] = o_y2


def _nms(s2, x12, y12, x22, y22, cnt2):
    out = jax.ShapeDtypeStruct((B * C_FG, TOPK), jnp.float32)
    return pl.pallas_call(
        _nms_body,
        out_shape=[out, out, out, out, out],
    )(s2, x12, y12, x22, y22, cnt2)


def kernel(loc_data, conf_data, prior_data):
    loc_t = jnp.transpose(loc_data, (0, 2, 1))
    prior_t = jnp.transpose(prior_data, (1, 0))
    conf3 = conf_data.reshape(B, P, C_ALL)

    dec, conf_t = _decode(loc_t, prior_t, conf3)
    s_c, x1_c, y1_c, x2_c, y2_c, cnts = _sc_compact(conf_t, dec)

    cnt2 = cnts[:, :, :CPW].reshape(B * C_FG, 1)
    outs = _nms(
        s_c.reshape(B * C_FG, CAP),
        x1_c.reshape(B * C_FG, CAP),
        y1_c.reshape(B * C_FG, CAP),
        x2_c.reshape(B * C_FG, CAP),
        y2_c.reshape(B * C_FG, CAP),
        cnt2,
    )
    planes = [o.reshape(B, C_FG, TOPK) for o in outs]
    per = jnp.stack(planes, axis=-1)
    bg = jnp.zeros((B, 1, TOPK, 5), jnp.float32)
    return jnp.concatenate([bg, per], axis=1)


# speedup vs baseline: 1.9049x; 1.0931x over previous
import functools

import jax
import jax.numpy as jnp
from jax import lax
from jax.experimental import pallas as pl
from jax.experimental.pallas import tpu as pltpu
from jax.experimental.pallas import tpu_sc as plsc

B = 4
P = 20000
C_ALL = 81
C_FG = 80
TOPK = 200
PRE = 500
CAP = 512
CLAMP = 527
PAD_CAP = 640
CONF_T = 0.98
NMS_T = 0.45
VAR0 = 0.1
VAR1 = 0.2

NC = 2
NS = 16
NW = NC * NS
WPB = NW // B
CPW = C_FG // WPB
LANES = 16


def _decode_body(loc_ref, prior_ref, conf_ref, out_ref, conft_ref):
    cx = prior_ref[0, :]
    cy = prior_ref[1, :]
    w = prior_ref[2, :]
    h = prior_ref[3, :]
    lx = loc_ref[0, 0, :]
    ly = loc_ref[0, 1, :]
    lw = loc_ref[0, 2, :]
    lh = loc_ref[0, 3, :]
    x = cx + lx * VAR0 * w
    y = cy + ly * VAR0 * h
    wd = w * jnp.exp(lw * VAR1)
    ht = h * jnp.exp(lh * VAR1)
    out_ref[0, 0, :] = x - wd / 2.0
    out_ref[0, 1, :] = y - ht / 2.0
    out_ref[0, 2, :] = x + wd / 2.0
    out_ref[0, 3, :] = y + ht / 2.0
    conft_ref[0] = jnp.transpose(conf_ref[0], (1, 0))[1:, :]


def _decode(loc_t, prior_t, conf3):
    return pl.pallas_call(
        _decode_body,
        grid=(B,),
        in_specs=[
            pl.BlockSpec((1, 4, P), lambda b: (b, 0, 0)),
            pl.BlockSpec((4, P), lambda b: (0, 0)),
            pl.BlockSpec((1, P, C_ALL), lambda b: (b, 0, 0)),
        ],
        out_specs=[
            pl.BlockSpec((1, 4, P), lambda b: (b, 0, 0)),
            pl.BlockSpec((1, C_FG, P), lambda b: (b, 0, 0)),
        ],
        out_shape=[
            jax.ShapeDtypeStruct((B, 4, P), jnp.float32),
            jax.ShapeDtypeStruct((B, C_FG, P), jnp.float32),
        ],
    )(loc_t, prior_t, conf3)


def _sc_body(conf_hbm, dec_hbm, s_out, x1_out, y1_out, x2_out, y2_out, cnt_out,
             dec_v, sc_v, cs_v, ci_v, cc_v, cn_v):
    wid = lax.axis_index("s") * NC + lax.axis_index("c")
    b = wid // WPB
    s8 = wid % WPB

    pltpu.sync_copy(dec_hbm.at[b], dec_v)

    lane_iota = lax.iota(jnp.int32, LANES)
    zf16 = jnp.zeros((LANES,), jnp.float32)
    zi16 = jnp.zeros((LANES,), jnp.int32)
    counts_vec = zi16

    for j in range(CPW):
        c = s8 * CPW + j
        pltpu.sync_copy(conf_hbm.at[b, c], sc_v)

        def _zero(g, _):
            off = pl.multiple_of(g * LANES, LANES)
            cs_v[pl.ds(off, LANES)] = zf16
            ci_v[pl.ds(off, LANES)] = zi16
            return 0

        lax.fori_loop(0, PAD_CAP // LANES, _zero, 0)

        UNROLL = 5

        def _compact(k, carry):
            offm1, piv = carry
            src = pl.multiple_of(k * (UNROLL * LANES), UNROLL * LANES)
            vs = [sc_v[pl.ds(src + u * LANES, LANES)] for u in range(UNROLL)]
            ms = [v > CONF_T for v in vs]
            cs = [plsc.cumsum(m.astype(jnp.int32)) for m in ms]
            pcs = [plsc.all_reduce_population_count(m) for m in ms]
            offc = jnp.minimum(offm1, CLAMP)
            run = offc
            for u in range(UNROLL):
                pos = run + cs[u]
                plsc.store_scatter(cs_v, [pos], vs[u], mask=ms[u])
                plsc.store_scatter(ci_v, [pos], piv + u * LANES, mask=ms[u])
                run = run + pcs[u]
            total = pcs[0]
            for u in range(1, UNROLL):
                total = total + pcs[u]
            return offm1 + total, piv + UNROLL * LANES

        offm1, _ = lax.fori_loop(
            0, P // (UNROLL * LANES), _compact,
            (jnp.full((LANES,), -1, jnp.int32), lane_iota))
        cnt_vec = offm1 + 1

        def _gather(g, _):
            off = pl.multiple_of(g * LANES, LANES)
            idxs = ci_v[pl.ds(off, LANES)]
            for p in range(4):
                pv = plsc.load_gather(
                    dec_v, [jnp.full((LANES,), p, jnp.int32), idxs])
                cc_v[p, pl.ds(off, LANES)] = pv
            return 0

        lax.fori_loop(0, CAP // LANES, _gather, 0)

        pltpu.sync_copy(cs_v.at[pl.ds(0, CAP)], s_out.at[b, c])
        pltpu.sync_copy(cc_v.at[0, pl.ds(0, CAP)], x1_out.at[b, c])
        pltpu.sync_copy(cc_v.at[1, pl.ds(0, CAP)], y1_out.at[b, c])
        pltpu.sync_copy(cc_v.at[2, pl.ds(0, CAP)], x2_out.at[b, c])
        pltpu.sync_copy(cc_v.at[3, pl.ds(0, CAP)], y2_out.at[b, c])

        counts_vec = counts_vec + jnp.where(lane_iota == j, cnt_vec, 0)

    cn_v[...] = counts_vec
    pltpu.sync_copy(cn_v, cnt_out.at[b, s8])


def _sc_compact(conf_t, dec):
    buf = jax.ShapeDtypeStruct((B, C_FG, CAP), jnp.float32)
    run = pl.kernel(
        _sc_body,
        out_type=[buf, buf, buf, buf, buf,
                  jax.ShapeDtypeStruct((B, WPB, LANES), jnp.int32)],
        mesh=plsc.VectorSubcoreMesh(core_axis_name="c", subcore_axis_name="s"),
        compiler_params=pltpu.CompilerParams(needs_layout_passes=False),
        scratch_types=[
            pltpu.VMEM((4, P), jnp.float32),
            pltpu.VMEM((P,), jnp.float32),
            pltpu.VMEM((PAD_CAP,), jnp.float32),
            pltpu.VMEM((PAD_CAP,), jnp.int32),
            pltpu.VMEM((4, PAD_CAP), jnp.float32),
            pltpu.VMEM((LANES,), jnp.int32),
        ],
    )
    return run(conf_t, dec)


def _nms_body(s_ref, x1_ref, y1_ref, x2_ref, y2_ref, cnt_ref,
              os_ref, ox1_ref, oy1_ref, ox2_ref, oy2_ref):
    R = B * C_FG
    s0 = s_ref[...]
    x1 = x1_ref[...]
    y1 = y1_ref[...]
    x2 = x2_ref[...]
    y2 = y2_ref[...]
    cnt = cnt_ref[...]

    need = cnt > PRE

    def _bs(_, lohi):
        lo, hi = lohi
        mid = (lo + hi) * 0.5
        cgt = jnp.sum((s0 > mid).astype(jnp.int32), axis=1, keepdims=True)
        ge = cgt >= PRE
        return jnp.where(ge, mid, lo), jnp.where(ge, hi, mid)

    lo, _ = lax.fori_loop(
        0, 30, _bs,
        (jnp.full((R, 1), CONF_T, jnp.float32),
         jnp.full((R, 1), 1.0, jnp.float32)))
    s0 = jnp.where(need & (s0 <= lo), 0.0, s0)

    area = jnp.maximum(x2 - x1, 0.0) * jnp.maximum(y2 - y1, 0.0)
    iota2 = lax.broadcasted_iota(jnp.int32, (R, CAP), 1)
    iota_t = lax.broadcasted_iota(jnp.int32, (R, TOPK), 1)
    zout = jnp.zeros((R, TOPK), jnp.float32)

    def _step(t, carry):
        s, o_s, o_x1, o_y1, o_x2, o_y2 = carry
        best = jnp.max(s, axis=1, keepdims=True)
        iseq = s == best
        idx = jnp.min(jnp.where(iseq, iota2, CAP), axis=1, keepdims=True)
        onehot = iseq & (iota2 == idx)
        ohf = onehot.astype(jnp.float32)
        bx1 = jnp.sum(x1 * ohf, axis=1, keepdims=True)
        by1 = jnp.sum(y1 * ohf, axis=1, keepdims=True)
        bx2 = jnp.sum(x2 * ohf, axis=1, keepdims=True)
        by2 = jnp.sum(y2 * ohf, axis=1, keepdims=True)
        barea = jnp.maximum(bx2 - bx1, 0.0) * jnp.maximum(by2 - by1, 0.0)

        ltx = jnp.maximum(x1, bx1)
        lty = jnp.maximum(y1, by1)
        rbx = jnp.minimum(x2, bx2)
        rby = jnp.minimum(y2, by2)
        inter = jnp.maximum(rbx - ltx, 0.0) * jnp.maximum(rby - lty, 0.0)
        union = barea + area - inter
        iou = inter / jnp.maximum(union, 1e-9)
        sup = iou > NMS_T

        valid = best > 0.0
        tm = iota_t == t
        o_s = jnp.where(tm & valid, best, o_s)
        o_x1 = jnp.where(tm & valid, bx1, o_x1)
        o_y1 = jnp.where(tm & valid, by1, o_y1)
        o_x2 = jnp.where(tm & valid, bx2, o_x2)
        o_y2 = jnp.where(tm & valid, by2, o_y2)
        s = jnp.where(sup | onehot, 0.0, s)
        return s, o_s, o_x1, o_y1, o_x2, o_y2

    _, o_s, o_x1, o_y1, o_x2, o_y2 = lax.fori_loop(
        0, TOPK, _step, (s0, zout, zout, zout, zout, zout))
    os_ref[...] = o_s
    ox1_ref[...] = o_x1
    oy1_ref[...] = o_y1
    ox2_ref[...] = o_x2
    oy2_ref[...] = o_y2


def _nms(s2, x12, y12, x22, y22, cnt2):
    out = jax.ShapeDtypeStruct((B * C_FG, TOPK), jnp.float32)
    return pl.pallas_call(
        _nms_body,
        out_shape=[out, out, out, out, out],
    )(s2, x12, y12, x22, y22, cnt2)


def kernel(loc_data, conf_data, prior_data):
    loc_t = jnp.transpose(loc_data, (0, 2, 1))
    prior_t = jnp.transpose(prior_data, (1, 0))
    conf3 = conf_data.reshape(B, P, C_ALL)

    dec, conf_t = _decode(loc_t, prior_t, conf3)
    s_c, x1_c, y1_c, x2_c, y2_c, cnts = _sc_compact(conf_t, dec)

    cnt2 = cnts[:, :, :CPW].reshape(B * C_FG, 1)
    outs = _nms(
        s_c.reshape(B * C_FG, CAP),
        x1_c.reshape(B * C_FG, CAP),
        y1_c.reshape(B * C_FG, CAP),
        x2_c.reshape(B * C_FG, CAP),
        y2_c.reshape(B * C_FG, CAP),
        cnt2,
    )
    planes = [o.reshape(B, C_FG, TOPK) for o in outs]
    per = jnp.stack(planes, axis=-1)
    bg = jnp.zeros((B, 1, TOPK, 5), jnp.float32)
    return jnp.concatenate([bg, per], axis=1)


# speedup vs baseline: 1.9706x; 1.0345x over previous
import functools

import jax
import jax.numpy as jnp
from jax import lax
from jax.experimental import pallas as pl
from jax.experimental.pallas import tpu as pltpu
from jax.experimental.pallas import tpu_sc as plsc

B = 4
P = 20000
C_ALL = 81
C_FG = 80
TOPK = 200
PRE = 500
CAP = 512
CLAMP = 527
PAD_CAP = 704
CONF_T = 0.98
NMS_T = 0.45
VAR0 = 0.1
VAR1 = 0.2

NC = 2
NS = 16
NW = NC * NS
WPB = NW // B
CPW = C_FG // WPB
LANES = 16


def _decode_body(loc_ref, prior_ref, conf_ref, out_ref, conft_ref):
    cx = prior_ref[0, :]
    cy = prior_ref[1, :]
    w = prior_ref[2, :]
    h = prior_ref[3, :]
    lx = loc_ref[0, 0, :]
    ly = loc_ref[0, 1, :]
    lw = loc_ref[0, 2, :]
    lh = loc_ref[0, 3, :]
    x = cx + lx * VAR0 * w
    y = cy + ly * VAR0 * h
    wd = w * jnp.exp(lw * VAR1)
    ht = h * jnp.exp(lh * VAR1)
    out_ref[0, 0, :] = x - wd / 2.0
    out_ref[0, 1, :] = y - ht / 2.0
    out_ref[0, 2, :] = x + wd / 2.0
    out_ref[0, 3, :] = y + ht / 2.0
    conft_ref[0] = jnp.transpose(conf_ref[0], (1, 0))[1:, :]


def _decode(loc_t, prior_t, conf3):
    return pl.pallas_call(
        _decode_body,
        grid=(B,),
        in_specs=[
            pl.BlockSpec((1, 4, P), lambda b: (b, 0, 0)),
            pl.BlockSpec((4, P), lambda b: (0, 0)),
            pl.BlockSpec((1, P, C_ALL), lambda b: (b, 0, 0)),
        ],
        out_specs=[
            pl.BlockSpec((1, 4, P), lambda b: (b, 0, 0)),
            pl.BlockSpec((1, C_FG, P), lambda b: (b, 0, 0)),
        ],
        out_shape=[
            jax.ShapeDtypeStruct((B, 4, P), jnp.float32),
            jax.ShapeDtypeStruct((B, C_FG, P), jnp.float32),
        ],
    )(loc_t, prior_t, conf3)


def _sc_body(conf_hbm, dec_hbm, s_out, x1_out, y1_out, x2_out, y2_out, cnt_out,
             dec_v, sc_v, cs_v, ci_v, cc_v, cn_v):
    wid = lax.axis_index("s") * NC + lax.axis_index("c")
    b = wid // WPB
    s8 = wid % WPB

    pltpu.sync_copy(dec_hbm.at[b], dec_v)

    lane_iota = lax.iota(jnp.int32, LANES)
    zf16 = jnp.zeros((LANES,), jnp.float32)
    zi16 = jnp.zeros((LANES,), jnp.int32)
    counts_vec = zi16

    for j in range(CPW):
        c = s8 * CPW + j
        pltpu.sync_copy(conf_hbm.at[b, c], sc_v)

        def _zero(g, _):
            off = pl.multiple_of(g * LANES, LANES)
            cs_v[pl.ds(off, LANES)] = zf16
            ci_v[pl.ds(off, LANES)] = zi16
            return 0

        lax.fori_loop(0, PAD_CAP // LANES, _zero, 0)

        UNROLL = 10

        def _compact(k, carry):
            offm1, piv = carry
            src = pl.multiple_of(k * (UNROLL * LANES), UNROLL * LANES)
            vs = [sc_v[pl.ds(src + u * LANES, LANES)] for u in range(UNROLL)]
            ms = [v > CONF_T for v in vs]
            cs = [plsc.cumsum(m.astype(jnp.int32)) for m in ms]
            pcs = [plsc.all_reduce_population_count(m) for m in ms]
            offc = jnp.minimum(offm1, CLAMP)
            run = offc
            for u in range(UNROLL):
                pos = run + cs[u]
                plsc.store_scatter(cs_v, [pos], vs[u], mask=ms[u])
                plsc.store_scatter(ci_v, [pos], piv + u * LANES, mask=ms[u])
                run = run + pcs[u]
            total = pcs[0]
            for u in range(1, UNROLL):
                total = total + pcs[u]
            return offm1 + total, piv + UNROLL * LANES

        offm1, _ = lax.fori_loop(
            0, P // (UNROLL * LANES), _compact,
            (jnp.full((LANES,), -1, jnp.int32), lane_iota))
        cnt_vec = offm1 + 1

        def _gather(g, _):
            off = pl.multiple_of(g * LANES, LANES)
            idxs = ci_v[pl.ds(off, LANES)]
            for p in range(4):
                pv = plsc.load_gather(
                    dec_v, [jnp.full((LANES,), p, jnp.int32), idxs])
                cc_v[p, pl.ds(off, LANES)] = pv
            return 0

        lax.fori_loop(0, CAP // LANES, _gather, 0)

        pltpu.sync_copy(cs_v.at[pl.ds(0, CAP)], s_out.at[b, c])
        pltpu.sync_copy(cc_v.at[0, pl.ds(0, CAP)], x1_out.at[b, c])
        pltpu.sync_copy(cc_v.at[1, pl.ds(0, CAP)], y1_out.at[b, c])
        pltpu.sync_copy(cc_v.at[2, pl.ds(0, CAP)], x2_out.at[b, c])
        pltpu.sync_copy(cc_v.at[3, pl.ds(0, CAP)], y2_out.at[b, c])

        counts_vec = counts_vec + jnp.where(lane_iota == j, cnt_vec, 0)

    cn_v[...] = counts_vec
    pltpu.sync_copy(cn_v, cnt_out.at[b, s8])


def _sc_compact(conf_t, dec):
    buf = jax.ShapeDtypeStruct((B, C_FG, CAP), jnp.float32)
    run = pl.kernel(
        _sc_body,
        out_type=[buf, buf, buf, buf, buf,
                  jax.ShapeDtypeStruct((B, WPB, LANES), jnp.int32)],
        mesh=plsc.VectorSubcoreMesh(core_axis_name="c", subcore_axis_name="s"),
        compiler_params=pltpu.CompilerParams(needs_layout_passes=False),
        scratch_types=[
            pltpu.VMEM((4, P), jnp.float32),
            pltpu.VMEM((P,), jnp.float32),
            pltpu.VMEM((PAD_CAP,), jnp.float32),
            pltpu.VMEM((PAD_CAP,), jnp.int32),
            pltpu.VMEM((4, PAD_CAP), jnp.float32),
            pltpu.VMEM((LANES,), jnp.int32),
        ],
    )
    return run(conf_t, dec)


def _nms_body(s_ref, x1_ref, y1_ref, x2_ref, y2_ref, cnt_ref,
              os_ref, ox1_ref, oy1_ref, ox2_ref, oy2_ref):
    R = B * C_FG
    s0 = s_ref[...]
    x1 = x1_ref[...]
    y1 = y1_ref[...]
    x2 = x2_ref[...]
    y2 = y2_ref[...]
    cnt = cnt_ref[...]

    need = cnt > PRE

    def _bs(_, lohi):
        lo, hi = lohi
        mid = (lo + hi) * 0.5
        cgt = jnp.sum((s0 > mid).astype(jnp.int32), axis=1, keepdims=True)
        ge = cgt >= PRE
        return jnp.where(ge, mid, lo), jnp.where(ge, hi, mid)

    lo, _ = lax.fori_loop(
        0, 30, _bs,
        (jnp.full((R, 1), CONF_T, jnp.float32),
         jnp.full((R, 1), 1.0, jnp.float32)))
    s0 = jnp.where(need & (s0 <= lo), 0.0, s0)

    area = jnp.maximum(x2 - x1, 0.0) * jnp.maximum(y2 - y1, 0.0)
    iota2 = lax.broadcasted_iota(jnp.int32, (R, CAP), 1)
    iota_t = lax.broadcasted_iota(jnp.int32, (R, TOPK), 1)
    zout = jnp.zeros((R, TOPK), jnp.float32)

    def _step(t, carry):
        s, o_s, o_x1, o_y1, o_x2, o_y2 = carry
        best = jnp.max(s, axis=1, keepdims=True)
        iseq = s == best
        idx = jnp.min(jnp.where(iseq, iota2, CAP), axis=1, keepdims=True)
        onehot = iseq & (iota2 == idx)
        ohf = onehot.astype(jnp.float32)
        bx1 = jnp.sum(x1 * ohf, axis=1, keepdims=True)
        by1 = jnp.sum(y1 * ohf, axis=1, keepdims=True)
        bx2 = jnp.sum(x2 * ohf, axis=1, keepdims=True)
        by2 = jnp.sum(y2 * ohf, axis=1, keepdims=True)
        barea = jnp.maximum(bx2 - bx1, 0.0) * jnp.maximum(by2 - by1, 0.0)

        ltx = jnp.maximum(x1, bx1)
        lty = jnp.maximum(y1, by1)
        rbx = jnp.minimum(x2, bx2)
        rby = jnp.minimum(y2, by2)
        inter = jnp.maximum(rbx - ltx, 0.0) * jnp.maximum(rby - lty, 0.0)
        union = barea + area - inter
        iou = inter / jnp.maximum(union, 1e-9)
        sup = iou > NMS_T

        valid = best > 0.0
        tm = iota_t == t
        o_s = jnp.where(tm & valid, best, o_s)
        o_x1 = jnp.where(tm & valid, bx1, o_x1)
        o_y1 = jnp.where(tm & valid, by1, o_y1)
        o_x2 = jnp.where(tm & valid, bx2, o_x2)
        o_y2 = jnp.where(tm & valid, by2, o_y2)
        s = jnp.where(sup | onehot, 0.0, s)
        return s, o_s, o_x1, o_y1, o_x2, o_y2

    _, o_s, o_x1, o_y1, o_x2, o_y2 = lax.fori_loop(
        0, TOPK, _step, (s0, zout, zout, zout, zout, zout))
    os_ref[...] = o_s
    ox1_ref[...] = o_x1
    oy1_ref[...] = o_y1
    ox2_ref[...] = o_x2
    oy2_ref[...] = o_y2


def _nms(s2, x12, y12, x22, y22, cnt2):
    out = jax.ShapeDtypeStruct((B * C_FG, TOPK), jnp.float32)
    return pl.pallas_call(
        _nms_body,
        out_shape=[out, out, out, out, out],
    )(s2, x12, y12, x22, y22, cnt2)


def kernel(loc_data, conf_data, prior_data):
    loc_t = jnp.transpose(loc_data, (0, 2, 1))
    prior_t = jnp.transpose(prior_data, (1, 0))
    conf3 = conf_data.reshape(B, P, C_ALL)

    dec, conf_t = _decode(loc_t, prior_t, conf3)
    s_c, x1_c, y1_c, x2_c, y2_c, cnts = _sc_compact(conf_t, dec)

    cnt2 = cnts[:, :, :CPW].reshape(B * C_FG, 1)
    outs = _nms(
        s_c.reshape(B * C_FG, CAP),
        x1_c.reshape(B * C_FG, CAP),
        y1_c.reshape(B * C_FG, CAP),
        x2_c.reshape(B * C_FG, CAP),
        y2_c.reshape(B * C_FG, CAP),
        cnt2,
    )
    planes = [o.reshape(B, C_FG, TOPK) for o in outs]
    per = jnp.stack(planes, axis=-1)
    bg = jnp.zeros((B, 1, TOPK, 5), jnp.float32)
    return jnp.concatenate([bg, per], axis=1)
